# baseline (speedup 1.0000x reference)
import functools

import jax
import jax.numpy as jnp
from jax import lax
from jax.experimental import pallas as pl
from jax.experimental.pallas import tpu as pltpu
from jax.experimental.pallas import tpu_sc as plsc

HID = 64
NFEAT = 16
NCLASS = 8
B = 4
MAXLEN = 512
KDIM = HID + 1
NGRP = 8
CPG = HID // NGRP
ROWS = 4 * CPG
WSLAB = NFEAT * KDIM * ROWS
BSLAB = NFEAT * ROWS
D = NFEAT * HID + HID
XPAD = 4 * MAXLEN + 16
MPAD = MAXLEN + 16


def _vfull(x):
    return jnp.full((16,), x, dtype=jnp.float32)


def _sigmoid(v):
    vc = jnp.minimum(jnp.maximum(v, -80.0), 80.0)
    return 1.0 / (1.0 + jnp.exp(-vc))


def _tanh(v):
    vc = jnp.minimum(jnp.maximum(v, -40.0), 40.0)
    e = jnp.exp(2.0 * vc)
    return (e - 1.0) / (e + 1.0)


def _gather(v, idx):
    return v.at[idx].get(mode="promise_in_bounds")


def _swap8(v):
    return _gather(v, jnp.bitwise_xor(lax.iota(jnp.int32, 16), 8))


def _butterfly(v, op):
    lanes = lax.iota(jnp.int32, 16)
    for sh in (1, 2, 4, 8):
        v = op(v, _gather(v, jnp.bitwise_xor(lanes, sh)))
    return v


def _allsum(v):
    return _butterfly(v, jnp.add)


def _allmax(v):
    return _butterfly(v, jnp.maximum)


def _lstm_sc(Xf, m_i32, len16, wt_f, bt_f, wd32, bd32, wo_f, bo_p):
    mesh = plsc.VectorSubcoreMesh(core_axis_name="c", subcore_axis_name="s")

    @functools.partial(
        pl.kernel,
        out_type=jax.ShapeDtypeStruct((B * 16,), jnp.float32),
        mesh=mesh,
        scratch_types=[
            pltpu.VMEM((WSLAB,), jnp.float32),
            pltpu.VMEM((BSLAB,), jnp.float32),
            pltpu.VMEM((XPAD,), jnp.float32),
            pltpu.VMEM((MPAD,), jnp.int32),
            pltpu.VMEM((16,), jnp.int32),
            pltpu.VMEM((B * 16,), jnp.int32),
            pltpu.VMEM((32,), jnp.float32),
            pltpu.VMEM((32,), jnp.float32),
            pltpu.VMEM((NCLASS * D,), jnp.float32),
            pltpu.VMEM((16,), jnp.float32),
            pltpu.VMEM((D + 16,), jnp.float32),
            pltpu.VMEM((16,), jnp.float32),
            pltpu.VMEM((HID,), jnp.float32),
            pltpu.VMEM_SHARED((2 * 2 * HID,), jnp.float32),
            pltpu.VMEM_SHARED((2 * HID,), jnp.float32),
        ],
    )
    def k(Xh, mh, lenh, wth, bth, wdh, bdh, woh, boh, outh,
          Wv, bv, xr, mv, lv, lv4, wdv, bdv, wov, bov, feat, st16, exin,
          exch, cex):
        cid = lax.axis_index("c")
        sid = lax.axis_index("s")
        bl = sid // NGRP
        gg = sid % NGRP
        b = cid * 2 + bl

        pltpu.sync_copy(wth.at[pl.ds(gg * WSLAB, WSLAB)], Wv)
        pltpu.sync_copy(bth.at[pl.ds(gg * BSLAB, BSLAB)], bv)
        pltpu.sync_copy(Xh.at[pl.ds(b * XPAD, XPAD)], xr)
        pltpu.sync_copy(mh.at[pl.ds(b * MPAD, MPAD)], mv)
        pltpu.sync_copy(lenh, lv4)
        pltpu.sync_copy(lenh.at[pl.ds(b * 16, 16)], lv)
        pltpu.sync_copy(wdh, wdv)
        pltpu.sync_copy(bdh, bdv)
        pltpu.sync_copy(woh, wov)
        pltpu.sync_copy(boh, bov)

        zeros = jnp.zeros((16,), jnp.float32)
        for i in range((D + 16) // 16):
            feat[pl.ds(i * 16, 16)] = zeros

        lenv = lv[...]
        lm = lv4[pl.ds(0, 16)]
        for q in range(1, B):
            lm = jnp.maximum(lm, lv4[pl.ds(q * 16, 16)])
        lmax = lm[0]

        lanes = lax.iota(jnp.int32, 16)
        lowhalf = lanes < 8

        bidx = [jnp.full((16,), l, jnp.int32) for l in range(16)]

        def step(j, carry):
            c, s, cnt, prev = carry
            tj = _gather(xr[pl.ds(j, 16)], bidx[0])
            xv = _gather(xr[pl.ds(2 * MAXLEN + j, 16)], bidx[0])
            dj = _gather(xr[pl.ds(3 * MAXLEN + j, 16)], bidx[0])
            fi = mv[pl.ds(j, 16)][0]

            jv = jnp.full((16,), j, dtype=jnp.int32)
            active = jv < lenv
            boundary = jnp.logical_and(
                active, jnp.logical_and(cnt > 0.0, tj != prev))
            cntmax = jnp.maximum(cnt, 1.0)
            c_b = jnp.where(boundary, s / cntmax, c)
            s_b = jnp.where(boundary, jnp.zeros_like(s), s)
            cnt_b = jnp.where(boundary, jnp.zeros_like(cnt), cnt)

            dval = (_gather(wdv[pl.ds(fi, 16)], bidx[0]) * dj
                    + _gather(bdv[pl.ds(fi, 16)], bidx[0]))
            decay = jnp.exp(-jnp.maximum(0.0, dval))

            hbase = HID + fi * HID
            hd = [decay * feat[pl.ds(hbase + r * 16, 16)] for r in range(4)]

            wbase = fi * (KDIM * ROWS)
            bbase = fi * ROWS
            p1 = [jnp.zeros((16,), jnp.float32) for _ in range(4)]
            p2 = [jnp.zeros((16,), jnp.float32) for _ in range(4)]
            p1[0] = bv[pl.ds(bbase, 16)] + xv * Wv[pl.ds(wbase, 16)]
            p2[0] = bv[pl.ds(bbase + 16, 16)] + xv * Wv[pl.ds(wbase + 16, 16)]
            for kk in range(1, KDIM):
                sk = _gather(hd[(kk - 1) // 16], bidx[(kk - 1) % 16])
                off = wbase + kk * ROWS
                q = kk % 4
                p1[q] = p1[q] + sk * Wv[pl.ds(off, 16)]
                p2[q] = p2[q] + sk * Wv[pl.ds(off + 16, 16)]
            a1 = (p1[0] + p1[1]) + (p1[2] + p1[3])
            a2 = (p2[0] + p2[1]) + (p2[2] + p2[3])

            s1 = _sigmoid(a1)
            g2 = jnp.where(lowhalf, _sigmoid(a2), _tanh(a2))
            gf_al = _swap8(s1)
            gc_al = _swap8(g2)
            new_c = gf_al * c_b + s1 * gc_al
            hnew = g2 * _tanh(new_c)

            own_old = feat[pl.ds(hbase + gg * CPG, 16)]
            pub = jnp.where(active, hnew, own_old)
            st16[...] = pub
            p = jnp.bitwise_and(j, 1)
            slot = p * (2 * HID) + bl * HID
            pltpu.sync_copy(st16.at[pl.ds(0, CPG)],
                            exch.at[pl.ds(slot + gg * CPG, CPG)])
            plsc.subcore_barrier()
            pltpu.sync_copy(exch.at[pl.ds(slot, HID)], exin)
            for r in range(4):
                feat[pl.ds(hbase + r * 16, 16)] = exin[pl.ds(r * 16, 16)]

            c2 = jnp.where(active, c_b, c)
            s2 = jnp.where(active, s_b + new_c, s)
            cnt2 = jnp.where(active, cnt_b + 1.0, cnt)
            prev2 = jnp.where(active, tj, prev)
            return (c2, s2, cnt2, prev2)

        init = (zeros, zeros, zeros, zeros)
        c_t, s_t, cnt_t, _ = lax.fori_loop(0, lmax, step, init)

        c_fin = jnp.where(cnt_t > 0.0, s_t / jnp.maximum(cnt_t, 1.0), c_t)
        st16[...] = c_fin
        pltpu.sync_copy(st16.at[pl.ds(0, CPG)],
                        cex.at[pl.ds(bl * HID + gg * CPG, CPG)])
        plsc.subcore_barrier()

        @pl.when(gg == 0)
        def _():
            pltpu.sync_copy(cex.at[pl.ds(bl * HID, HID)], exin)
            for r in range(4):
                feat[pl.ds(r * 16, 16)] = exin[pl.ds(r * 16, 16)]
            logit = bov[...]
            for i in range(NCLASS):
                a = jnp.zeros((16,), jnp.float32)
                for v in range(D // 16):
                    a = a + wov[pl.ds(i * D + v * 16, 16)] * feat[pl.ds(v * 16, 16)]
                onehot = jnp.where(lanes == i, 1.0, 0.0).astype(jnp.float32)
                logit = logit + _allsum(a) * onehot
            mx = _allmax(logit)
            e = jnp.exp(logit - mx)
            ssum = _allsum(e)
            st16[...] = e / ssum
            pltpu.sync_copy(st16, outh.at[pl.ds(b * 16, 16)])

    return k(Xf, m_i32, len16, wt_f, bt_f, wd32, bd32, wo_f, bo_p)


def kernel(X, lengths, W_layers, b_layers, W_decay, b_decay, W_out, b_out):
    Xf = jnp.pad(X.reshape(B, 4 * MAXLEN), ((0, 0), (0, 16))).reshape(-1)
    m_i32 = jnp.pad(X[:, 1].astype(jnp.int32), ((0, 0), (0, 16))).reshape(-1)
    len16 = jnp.tile(lengths.astype(jnp.int32)[:, None], (1, 16)).reshape(-1)
    wt = W_layers.reshape(NFEAT, 4, NGRP, CPG, KDIM)
    wt = wt.transpose(2, 0, 4, 1, 3).reshape(NGRP * WSLAB)
    bt = b_layers.reshape(NFEAT, 4, NGRP, CPG)
    bt = bt.transpose(2, 0, 1, 3).reshape(NGRP * BSLAB)
    wd32 = jnp.pad(W_decay, (0, 16))
    bd32 = jnp.pad(b_decay, (0, 16))
    wo_f = W_out.reshape(NCLASS * D)
    bo_p = jnp.concatenate([b_out, jnp.full((16 - NCLASS,), -1e30, jnp.float32)])
    out = _lstm_sc(Xf, m_i32, len16, wt, bt, wd32, bd32, wo_f, bo_p)
    return out.reshape(B, 16)[:, :NCLASS]


# speedup vs baseline: 1.0125x; 1.0125x over previous
import functools

import jax
import jax.numpy as jnp
from jax import lax
from jax.experimental import pallas as pl
from jax.experimental.pallas import tpu as pltpu
from jax.experimental.pallas import tpu_sc as plsc

HID = 64
NFEAT = 16
NCLASS = 8
B = 4
MAXLEN = 512
KDIM = HID + 1
NGRP = 8
CPG = HID // NGRP
ROWS = 4 * CPG
WSLAB = NFEAT * KDIM * ROWS
BSLAB = NFEAT * ROWS
D = NFEAT * HID + HID
XPAD = 4 * MAXLEN + 16
MPAD = MAXLEN + 16


def _vfull(x):
    return jnp.full((16,), x, dtype=jnp.float32)


def _sigmoid(v):
    vc = jnp.minimum(jnp.maximum(v, -80.0), 80.0)
    return 1.0 / (1.0 + jnp.exp(-vc))


def _tanh(v):
    vc = jnp.minimum(jnp.maximum(v, -40.0), 40.0)
    e = jnp.exp(2.0 * vc)
    return (e - 1.0) / (e + 1.0)


def _gather(v, idx):
    return v.at[idx].get(mode="promise_in_bounds")


def _swap8(v):
    return _gather(v, jnp.bitwise_xor(lax.iota(jnp.int32, 16), 8))


def _butterfly(v, op):
    lanes = lax.iota(jnp.int32, 16)
    for sh in (1, 2, 4, 8):
        v = op(v, _gather(v, jnp.bitwise_xor(lanes, sh)))
    return v


def _allsum(v):
    return _butterfly(v, jnp.add)


def _allmax(v):
    return _butterfly(v, jnp.maximum)


def _lstm_sc(Xf, m_i32, len16, wt_f, bt_f, wd32, bd32, wo_f, bo_p):
    mesh = plsc.VectorSubcoreMesh(core_axis_name="c", subcore_axis_name="s")

    @functools.partial(
        pl.kernel,
        out_type=jax.ShapeDtypeStruct((B * 16,), jnp.float32),
        mesh=mesh,
        scratch_types=[
            pltpu.VMEM((WSLAB,), jnp.float32),
            pltpu.VMEM((BSLAB,), jnp.float32),
            pltpu.VMEM((XPAD,), jnp.float32),
            pltpu.VMEM((MPAD,), jnp.int32),
            pltpu.VMEM((16,), jnp.int32),
            pltpu.VMEM((B * 16,), jnp.int32),
            pltpu.VMEM((32,), jnp.float32),
            pltpu.VMEM((32,), jnp.float32),
            pltpu.VMEM((NCLASS * D,), jnp.float32),
            pltpu.VMEM((16,), jnp.float32),
            pltpu.VMEM((D + 16,), jnp.float32),
            pltpu.VMEM((MPAD,), jnp.float32),
            pltpu.VMEM((16,), jnp.float32),
            pltpu.VMEM((HID,), jnp.float32),
            pltpu.VMEM_SHARED((2 * 2 * HID,), jnp.float32),
            pltpu.VMEM_SHARED((2 * HID,), jnp.float32),
        ],
    )
    def k(Xh, mh, lenh, wth, bth, wdh, bdh, woh, boh, outh,
          Wv, bv, xr, mv, lv, lv4, wdv, bdv, wov, bov, feat, decb, st16,
          exin, exch, cex):
        cid = lax.axis_index("c")
        sid = lax.axis_index("s")
        bl = sid // NGRP
        gg = sid % NGRP
        b = cid * 2 + bl

        pltpu.sync_copy(wth.at[pl.ds(gg * WSLAB, WSLAB)], Wv)
        pltpu.sync_copy(bth.at[pl.ds(gg * BSLAB, BSLAB)], bv)
        pltpu.sync_copy(Xh.at[pl.ds(b * XPAD, XPAD)], xr)
        pltpu.sync_copy(mh.at[pl.ds(b * MPAD, MPAD)], mv)
        pltpu.sync_copy(lenh, lv4)
        pltpu.sync_copy(lenh.at[pl.ds(b * 16, 16)], lv)
        pltpu.sync_copy(wdh, wdv)
        pltpu.sync_copy(bdh, bdv)
        pltpu.sync_copy(woh, wov)
        pltpu.sync_copy(boh, bov)

        zeros = jnp.zeros((16,), jnp.float32)
        for i in range((D + 16) // 16):
            feat[pl.ds(i * 16, 16)] = zeros

        lenv = lv[...]
        lm = lv4[pl.ds(0, 16)]
        for q in range(1, B):
            lm = jnp.maximum(lm, lv4[pl.ds(q * 16, 16)])
        lmax = lm[0]

        lanes = lax.iota(jnp.int32, 16)
        lowhalf = lanes < 8

        bidx = [jnp.full((16,), l, jnp.int32) for l in range(16)]

        wd_all = wdv[pl.ds(0, 16)]
        bd_all = bdv[pl.ds(0, 16)]
        for blk in range(MAXLEN // 16):
            fiv = mv[pl.ds(blk * 16, 16)]
            dl = xr[pl.ds(3 * MAXLEN + blk * 16, 16)]
            dv = _gather(wd_all, fiv) * dl + _gather(bd_all, fiv)
            decb[pl.ds(blk * 16, 16)] = jnp.exp(-jnp.maximum(0.0, dv))
        decb[pl.ds(MAXLEN, 16)] = zeros

        sel2 = jnp.where(lowhalf, -1.0, 2.0).astype(jnp.float32)
        one = jnp.ones((16,), jnp.float32)

        def step(j, carry):
            c, s, cnt, prev = carry
            tj = _gather(xr[pl.ds(j, 16)], bidx[0])
            xv = _gather(xr[pl.ds(2 * MAXLEN + j, 16)], bidx[0])
            dj = _gather(xr[pl.ds(3 * MAXLEN + j, 16)], bidx[0])
            fi = mv[pl.ds(j, 16)][0]

            jv = jnp.full((16,), j, dtype=jnp.int32)
            active = jv < lenv
            boundary = jnp.logical_and(
                active, jnp.logical_and(cnt > 0.0, tj != prev))
            cntmax = jnp.maximum(cnt, 1.0)
            c_b = jnp.where(boundary, s / cntmax, c)
            s_b = jnp.where(boundary, jnp.zeros_like(s), s)
            cnt_b = jnp.where(boundary, jnp.zeros_like(cnt), cnt)

            decay = _gather(decb[pl.ds(j, 16)], bidx[0])

            hbase = HID + fi * HID
            hd = [decay * feat[pl.ds(hbase + r * 16, 16)] for r in range(4)]

            wbase = fi * (KDIM * ROWS)
            bbase = fi * ROWS
            p1 = [jnp.zeros((16,), jnp.float32) for _ in range(4)]
            p2 = [jnp.zeros((16,), jnp.float32) for _ in range(4)]
            p1[0] = bv[pl.ds(bbase, 16)] + xv * Wv[pl.ds(wbase, 16)]
            p2[0] = bv[pl.ds(bbase + 16, 16)] + xv * Wv[pl.ds(wbase + 16, 16)]
            for kk in range(1, KDIM):
                sk = _gather(hd[(kk - 1) // 16], bidx[(kk - 1) % 16])
                off = wbase + kk * ROWS
                q = kk % 4
                p1[q] = p1[q] + sk * Wv[pl.ds(off, 16)]
                p2[q] = p2[q] + sk * Wv[pl.ds(off + 16, 16)]
            a1 = (p1[0] + p1[1]) + (p1[2] + p1[3])
            a2 = (p2[0] + p2[1]) + (p2[2] + p2[3])

            s1 = _sigmoid(a1)
            e2 = jnp.exp(sel2 * jnp.minimum(jnp.maximum(a2, -40.0), 40.0))
            g2 = jnp.where(lowhalf, one, e2 - 1.0) / (one + e2)
            gf_al = _swap8(s1)
            gc_al = _swap8(g2)
            new_c = gf_al * c_b + s1 * gc_al
            hnew = g2 * _tanh(new_c)

            own_old = feat[pl.ds(hbase + gg * CPG, 16)]
            pub = jnp.where(active, hnew, own_old)
            st16[...] = pub
            p = jnp.bitwise_and(j, 1)
            slot = p * (2 * HID) + bl * HID
            pltpu.sync_copy(st16.at[pl.ds(0, CPG)],
                            exch.at[pl.ds(slot + gg * CPG, CPG)])
            plsc.subcore_barrier()
            pltpu.sync_copy(exch.at[pl.ds(slot, HID)], exin)
            for r in range(4):
                feat[pl.ds(hbase + r * 16, 16)] = exin[pl.ds(r * 16, 16)]

            c2 = jnp.where(active, c_b, c)
            s2 = jnp.where(active, s_b + new_c, s)
            cnt2 = jnp.where(active, cnt_b + 1.0, cnt)
            prev2 = jnp.where(active, tj, prev)
            return (c2, s2, cnt2, prev2)

        init = (zeros, zeros, zeros, zeros)
        c_t, s_t, cnt_t, _ = lax.fori_loop(0, lmax, step, init)

        c_fin = jnp.where(cnt_t > 0.0, s_t / jnp.maximum(cnt_t, 1.0), c_t)
        st16[...] = c_fin
        pltpu.sync_copy(st16.at[pl.ds(0, CPG)],
                        cex.at[pl.ds(bl * HID + gg * CPG, CPG)])
        plsc.subcore_barrier()

        @pl.when(gg == 0)
        def _():
            pltpu.sync_copy(cex.at[pl.ds(bl * HID, HID)], exin)
            for r in range(4):
                feat[pl.ds(r * 16, 16)] = exin[pl.ds(r * 16, 16)]
            logit = bov[...]
            for i in range(NCLASS):
                a = jnp.zeros((16,), jnp.float32)
                for v in range(D // 16):
                    a = a + wov[pl.ds(i * D + v * 16, 16)] * feat[pl.ds(v * 16, 16)]
                onehot = jnp.where(lanes == i, 1.0, 0.0).astype(jnp.float32)
                logit = logit + _allsum(a) * onehot
            mx = _allmax(logit)
            e = jnp.exp(logit - mx)
            ssum = _allsum(e)
            st16[...] = e / ssum
            pltpu.sync_copy(st16, outh.at[pl.ds(b * 16, 16)])

    return k(Xf, m_i32, len16, wt_f, bt_f, wd32, bd32, wo_f, bo_p)


def kernel(X, lengths, W_layers, b_layers, W_decay, b_decay, W_out, b_out):
    Xf = jnp.pad(X.reshape(B, 4 * MAXLEN), ((0, 0), (0, 16))).reshape(-1)
    m_i32 = jnp.pad(X[:, 1].astype(jnp.int32), ((0, 0), (0, 16))).reshape(-1)
    len16 = jnp.tile(lengths.astype(jnp.int32)[:, None], (1, 16)).reshape(-1)
    wt = W_layers.reshape(NFEAT, 4, NGRP, CPG, KDIM)
    wt = wt.transpose(2, 0, 4, 1, 3).reshape(NGRP * WSLAB)
    bt = b_layers.reshape(NFEAT, 4, NGRP, CPG)
    bt = bt.transpose(2, 0, 1, 3).reshape(NGRP * BSLAB)
    wd32 = jnp.pad(W_decay, (0, 16))
    bd32 = jnp.pad(b_decay, (0, 16))
    wo_f = W_out.reshape(NCLASS * D)
    bo_p = jnp.concatenate([b_out, jnp.full((16 - NCLASS,), -1e30, jnp.float32)])
    out = _lstm_sc(Xf, m_i32, len16, wt, bt, wd32, bd32, wo_f, bo_p)
    return out.reshape(B, 16)[:, :NCLASS]
